# baseline (speedup 1.0000x reference)
import jax
import jax.numpy as jnp
from jax.experimental import pallas as pl
from jax.experimental.pallas import tpu as pltpu

_BM = 256
_NBUF = 4


def _mm_kernel(a_hbm, f_ref, o_ref, abuf, sems):
    nsteps = a_hbm.shape[0] // _BM

    def copy(slot, i):
        return pltpu.make_async_copy(
            a_hbm.at[pl.ds(i * _BM, _BM), :], abuf.at[slot], sems.at[slot])

    for b in range(_NBUF - 1):
        copy(b, b).start()
    for i in range(nsteps):
        nxt = i + _NBUF - 1
        if nxt < nsteps:
            copy(nxt % _NBUF, nxt).start()
        slot = i % _NBUF
        copy(slot, i).wait()
        o_ref[pl.ds(i * _BM, _BM), :] = jax.lax.dot_general(
            abuf[slot], f_ref[...],
            dimension_numbers=(((1,), (0,)), ((), ())),
            precision=jax.lax.Precision.DEFAULT,
            preferred_element_type=jnp.float32)


@jax.jit
def kernel(features, A):
    if features.ndim != 2:
        raise RuntimeError('the dimension of features should be 2')
    M, K = A.shape
    _, N = features.shape
    return pl.pallas_call(
        _mm_kernel,
        in_specs=[
            pl.BlockSpec(memory_space=pl.ANY),
            pl.BlockSpec(memory_space=pltpu.VMEM),
        ],
        out_specs=pl.BlockSpec(memory_space=pltpu.VMEM),
        out_shape=jax.ShapeDtypeStruct((M, N), jnp.float32),
        scratch_shapes=[
            pltpu.VMEM((_NBUF, _BM, K), jnp.float32),
            pltpu.SemaphoreType.DMA((_NBUF,)),
        ],
        compiler_params=pltpu.CompilerParams(
            vmem_limit_bytes=100 * 1024 * 1024,
        ),
    )(A, features)


# speedup vs baseline: 1.0129x; 1.0129x over previous
import jax
import jax.numpy as jnp
from jax.experimental import pallas as pl
from jax.experimental.pallas import tpu as pltpu


def _matmul_block(a_ref, f_ref, o_ref):
    o_ref[...] = jax.lax.dot_general(
        a_ref[...], f_ref[...],
        dimension_numbers=(((1,), (0,)), ((), ())),
        precision=jax.lax.Precision.DEFAULT,
        preferred_element_type=jnp.float32)


@jax.jit
def kernel(features, A):
    if features.ndim != 2:
        raise RuntimeError('the dimension of features should be 2')
    M, K = A.shape
    _, N = features.shape
    BM = 256
    return pl.pallas_call(
        _matmul_block,
        grid=(M // BM,),
        in_specs=[
            pl.BlockSpec((BM, K), lambda i: (i, 0)),
            pl.BlockSpec((K, N), lambda i: (0, 0)),
        ],
        out_specs=pl.BlockSpec((BM, N), lambda i: (i, 0)),
        out_shape=jax.ShapeDtypeStruct((M, N), jnp.float32),
        compiler_params=pltpu.CompilerParams(
            dimension_semantics=("parallel",),
        ),
    )(A, features.astype(jnp.bfloat16))


# speedup vs baseline: 1.0590x; 1.0455x over previous
import jax
import jax.numpy as jnp
from jax.experimental import pallas as pl
from jax.experimental.pallas import tpu as pltpu


def _matmul_block(a_ref, f_ref, o_ref):
    o_ref[...] = jax.lax.dot_general(
        a_ref[...], f_ref[...],
        dimension_numbers=(((1,), (0,)), ((), ())),
        precision=jax.lax.Precision.DEFAULT,
        preferred_element_type=jnp.float32)


@jax.jit
def kernel(features, A):
    if features.ndim != 2:
        raise RuntimeError('the dimension of features should be 2')
    M, K = A.shape
    _, N = features.shape
    BM = 256
    return pl.pallas_call(
        _matmul_block,
        grid=(M // BM,),
        in_specs=[
            pl.BlockSpec((BM, K), lambda i: (i, 0)),
            pl.BlockSpec((K, N), lambda i: (0, 0)),
        ],
        out_specs=pl.BlockSpec((BM, N), lambda i: (i, 0)),
        out_shape=jax.ShapeDtypeStruct((M, N), jnp.float32),
        compiler_params=pltpu.CompilerParams(
            dimension_semantics=("parallel",),
        ),
    )(A, features)
